# baseline (speedup 1.0000x reference)
import jax
import jax.numpy as jnp
from jax.experimental import pallas as pl
from jax.experimental.pallas import tpu as pltpu
from jax.experimental.pallas import tpu_sc as plsc

BATCH = 16384
DIM = 32
N_WORKERS = 32
CHUNK = BATCH // N_WORKERS
MLP_BLOCK = 2048


def _sc_gather(user_emb, item_emb, user_id, item_id):
    mesh = plsc.VectorSubcoreMesh(core_axis_name="core", subcore_axis_name="subcore")
    out_t = (
        jax.ShapeDtypeStruct((BATCH, DIM), jnp.float32),
        jax.ShapeDtypeStruct((BATCH, DIM), jnp.float32),
    )

    @pl.kernel(
        out_type=out_t,
        mesh=mesh,
        scratch_types=[
            pltpu.VMEM((CHUNK,), jnp.int32),
            pltpu.VMEM((CHUNK,), jnp.int32),
            pltpu.SemaphoreType.DMA,
            pltpu.SemaphoreType.DMA,
        ],
    )
    def gather_kernel(ue_hbm, ie_hbm, uid_hbm, iid_hbm, ue_out, ie_out,
                      uidx, iidx, usem, isem):
        core = jax.lax.axis_index("core")
        sub = jax.lax.axis_index("subcore")
        wid = core * 16 + sub
        base = wid * CHUNK

        pltpu.sync_copy(uid_hbm.at[pl.ds(base, CHUNK)], uidx)
        pltpu.sync_copy(iid_hbm.at[pl.ds(base, CHUNK)], iidx)

        @pl.loop(0, CHUNK, step=16)
        def _issue(c):
            uv = uidx[pl.ds(c, 16)]
            iv = iidx[pl.ds(c, 16)]
            for k in range(16):
                pltpu.async_copy(
                    ue_hbm.at[pl.ds(uv[k], 1), :],
                    ue_out.at[pl.ds(base + c + k, 1), :], usem)
                pltpu.async_copy(
                    ie_hbm.at[pl.ds(iv[k], 1), :],
                    ie_out.at[pl.ds(base + c + k, 1), :], isem)

        @pl.loop(0, CHUNK)
        def _drain(r):
            pltpu.make_async_copy(
                ue_hbm.at[pl.ds(0, 1), :], ue_out.at[pl.ds(0, 1), :], usem).wait()
            pltpu.make_async_copy(
                ie_hbm.at[pl.ds(0, 1), :], ie_out.at[pl.ds(0, 1), :], isem).wait()

    return gather_kernel(user_emb, item_emb, user_id, item_id)


def _mlp_body(ue_ref, ie_ref, w0a_ref, w0b_ref, b0_ref, w1_ref, b1_ref,
              w2_ref, b2_ref, wo_ref, bo_ref, o_ref):
    x = ue_ref[...] @ w0a_ref[...] + ie_ref[...] @ w0b_ref[...] + b0_ref[...]
    x = jnp.maximum(x, 0.0)
    x = jnp.maximum(x @ w1_ref[...] + b1_ref[...], 0.0)
    x = jnp.maximum(x @ w2_ref[...] + b2_ref[...], 0.0)
    o_ref[...] = jax.nn.sigmoid(x @ wo_ref[...] + bo_ref[...])


def _tc_mlp(ue, ie, W0, b0, W1, b1, W2, b2, Wout, bout):
    w0a = W0[:DIM]
    w0b = W0[DIM:]
    full = lambda shape: pl.BlockSpec(shape, lambda i: (0, 0))
    grid = (BATCH // MLP_BLOCK,)
    return pl.pallas_call(
        _mlp_body,
        grid=grid,
        in_specs=[
            pl.BlockSpec((MLP_BLOCK, DIM), lambda i: (i, 0)),
            pl.BlockSpec((MLP_BLOCK, DIM), lambda i: (i, 0)),
            full(w0a.shape),
            full(w0b.shape),
            full((1, b0.shape[0])),
            full(W1.shape),
            full((1, b1.shape[0])),
            full(W2.shape),
            full((1, b2.shape[0])),
            full(Wout.shape),
            full((1, bout.shape[0])),
        ],
        out_specs=pl.BlockSpec((MLP_BLOCK, 1), lambda i: (i, 0)),
        out_shape=jax.ShapeDtypeStruct((BATCH, 1), jnp.float32),
    )(ue, ie, w0a, w0b, b0.reshape(1, -1), W1, b1.reshape(1, -1),
      W2, b2.reshape(1, -1), Wout, bout.reshape(1, -1))


def kernel(user_id, item_id, user_emb, item_emb, W0, b0, W1, b1, W2, b2, Wout, bout):
    user_id = user_id.astype(jnp.int32)
    item_id = item_id.astype(jnp.int32)
    ue, ie = _sc_gather(user_emb, item_emb, user_id, item_id)
    return _tc_mlp(ue, ie, W0, b0, W1, b1, W2, b2, Wout, bout)


# speedup vs baseline: 1.9141x; 1.9141x over previous
import jax
import jax.numpy as jnp
from jax.experimental import pallas as pl
from jax.experimental.pallas import tpu as pltpu
from jax.experimental.pallas import tpu_sc as plsc

BATCH = 16384
DIM = 32
N_TABLE = 1000000
PACK = 128 // DIM
WIDE = 128
TR_ROWS = 1024
TR_COLS = TR_ROWS * PACK
N_FULL = N_TABLE // TR_COLS
TAIL = N_TABLE - N_FULL * TR_COLS
N_PACKED = (N_FULL + 1) * TR_ROWS
GATHER_WINDOW = 128
MLP_BLOCK = 2048


def _pack_body(a_ref, b_ref, at_ref, bt_ref, oa_ref, ob_ref):
    eye = jnp.eye(DIM, dtype=jnp.float32)
    is_tail = pl.program_id(0) == N_FULL
    for src, tail, dst in ((a_ref, at_ref, oa_ref), (b_ref, bt_ref, ob_ref)):
        x = jnp.where(is_tail, tail[...], src[...])
        parts = [
            jax.lax.dot_general(
                x[:, a * TR_ROWS:(a + 1) * TR_ROWS], eye,
                (((0,), (0,)), ((), ())),
                preferred_element_type=jnp.float32)
            for a in range(PACK)
        ]
        dst[...] = jnp.concatenate(parts, axis=1)


def _tc_pack(ue_t, ie_t):
    ue_tail = jnp.pad(ue_t[:, N_FULL * TR_COLS:], ((0, 0), (0, TR_COLS - TAIL)))
    ie_tail = jnp.pad(ie_t[:, N_FULL * TR_COLS:], ((0, 0), (0, TR_COLS - TAIL)))
    grid = (N_FULL + 1,)
    out_sds = jax.ShapeDtypeStruct((N_PACKED, WIDE), jnp.float32)
    main_spec = pl.BlockSpec((DIM, TR_COLS), lambda i: (0, jnp.minimum(i, N_FULL - 1)))
    tail_spec = pl.BlockSpec((DIM, TR_COLS), lambda i: (0, 0))
    return pl.pallas_call(
        _pack_body,
        grid=grid,
        in_specs=[main_spec, main_spec, tail_spec, tail_spec],
        out_specs=[
            pl.BlockSpec((TR_ROWS, WIDE), lambda i: (i, 0)),
            pl.BlockSpec((TR_ROWS, WIDE), lambda i: (i, 0)),
        ],
        out_shape=(out_sds, out_sds),
    )(ue_t, ie_t, ue_tail, ie_tail)


def _sc_gather(ue_wide, ie_wide, uid4, iid4):
    mesh = plsc.VectorSubcoreMesh(core_axis_name="core", subcore_axis_name="subcore")
    uid = uid4.reshape(1, BATCH)
    iid = iid4.reshape(1, BATCH)
    out_t = (
        jax.ShapeDtypeStruct((BATCH, WIDE), jnp.float32),
        jax.ShapeDtypeStruct((BATCH, WIDE), jnp.float32),
    )

    @pl.kernel(out_type=out_t, mesh=mesh)
    def gather_kernel(ue_hbm, ie_hbm, uid_hbm, iid_hbm, ue_out, ie_out):
        def body(uid_vmem, iid_vmem, ue_vmem, ie_vmem):
            pltpu.sync_copy(ue_hbm.at[uid_vmem.at[0]], ue_vmem)
            pltpu.sync_copy(ie_hbm.at[iid_vmem.at[0]], ie_vmem)

        pltpu.emit_pipeline(
            body,
            grid=(BATCH // GATHER_WINDOW,),
            in_specs=[
                pl.BlockSpec((1, GATHER_WINDOW), lambda i: (0, i)),
                pl.BlockSpec((1, GATHER_WINDOW), lambda i: (0, i)),
            ],
            out_specs=[
                pl.BlockSpec((GATHER_WINDOW, WIDE), lambda i: (i, 0)),
                pl.BlockSpec((GATHER_WINDOW, WIDE), lambda i: (i, 0)),
            ],
            core_axis_name=("core", "subcore"),
            dimension_semantics=(pltpu.PARALLEL,),
        )(uid_hbm, iid_hbm, ue_out, ie_out)

    return gather_kernel(ue_wide, ie_wide, uid, iid)


def _select_quadrant(wide, q):
    out = jnp.zeros((wide.shape[0], DIM), jnp.float32)
    for k in range(PACK):
        mask = (q == k).astype(jnp.float32)
        out = out + mask * wide[:, k * DIM:(k + 1) * DIM]
    return out


def _mlp_body(wu_ref, wi_ref, qu_ref, qi_ref, w0a_ref, w0b_ref, b0_ref,
              w1_ref, b1_ref, w2_ref, b2_ref, wo_ref, bo_ref, o_ref):
    ue = _select_quadrant(wu_ref[...], qu_ref[...])
    ie = _select_quadrant(wi_ref[...], qi_ref[...])
    x = ue @ w0a_ref[...] + ie @ w0b_ref[...] + b0_ref[...]
    x = jnp.maximum(x, 0.0)
    x = jnp.maximum(x @ w1_ref[...] + b1_ref[...], 0.0)
    x = jnp.maximum(x @ w2_ref[...] + b2_ref[...], 0.0)
    o_ref[...] = jax.nn.sigmoid(x @ wo_ref[...] + bo_ref[...])


def _tc_mlp(wu, wi, qu, qi, W0, b0, W1, b1, W2, b2, Wout, bout):
    w0a = W0[:DIM]
    w0b = W0[DIM:]
    full = lambda shape: pl.BlockSpec(shape, lambda i: (0, 0))
    grid = (BATCH // MLP_BLOCK,)
    return pl.pallas_call(
        _mlp_body,
        grid=grid,
        in_specs=[
            pl.BlockSpec((MLP_BLOCK, WIDE), lambda i: (i, 0)),
            pl.BlockSpec((MLP_BLOCK, WIDE), lambda i: (i, 0)),
            pl.BlockSpec((MLP_BLOCK, 1), lambda i: (i, 0)),
            pl.BlockSpec((MLP_BLOCK, 1), lambda i: (i, 0)),
            full(w0a.shape),
            full(w0b.shape),
            full((1, b0.shape[0])),
            full(W1.shape),
            full((1, b1.shape[0])),
            full(W2.shape),
            full((1, b2.shape[0])),
            full(Wout.shape),
            full((1, bout.shape[0])),
        ],
        out_specs=pl.BlockSpec((MLP_BLOCK, 1), lambda i: (i, 0)),
        out_shape=jax.ShapeDtypeStruct((BATCH, 1), jnp.float32),
    )(wu, wi, qu, qi, w0a, w0b, b0.reshape(1, -1), W1, b1.reshape(1, -1),
      W2, b2.reshape(1, -1), Wout, bout.reshape(1, -1))


def kernel(user_id, item_id, user_emb, item_emb, W0, b0, W1, b1, W2, b2, Wout, bout):
    user_id = user_id.astype(jnp.int32)
    item_id = item_id.astype(jnp.int32)
    ue_wide, ie_wide = _tc_pack(user_emb.T, item_emb.T)
    uw = TR_ROWS * (user_id // TR_COLS) + user_id % TR_ROWS
    iw = TR_ROWS * (item_id // TR_COLS) + item_id % TR_ROWS
    wu, wi = _sc_gather(ue_wide, ie_wide, uw, iw)
    qu = ((user_id // TR_ROWS) % PACK).reshape(BATCH, 1)
    qi = ((item_id // TR_ROWS) % PACK).reshape(BATCH, 1)
    return _tc_mlp(wu, wi, qu, qi, W0, b0, W1, b1, W2, b2, Wout, bout)
